# Initial kernel scaffold; baseline (speedup 1.0000x reference)
import jax
import jax.numpy as jnp
from jax.experimental import pallas as pl


def kernel(x0, x1, edge_index0, edge_index1, batch0, batch1, params, train):
    raise NotImplementedError("write your pallas kernel here")


# speedup vs baseline: 2.4198x; 2.4198x over previous
import functools

import jax
import jax.numpy as jnp
from jax import lax
from jax.experimental import pallas as pl
from jax.experimental.pallas import tpu as pltpu

N_NODES = 10000
N_GRAPHS = 64


def _mm2_body(h_ref, wl_ref, wr_ref, xl_ref, xr_ref):
    h = h_ref[0]
    xl_ref[0] = jnp.dot(h, wl_ref[0], preferred_element_type=jnp.float32)
    xr_ref[0] = jnp.dot(h, wr_ref[0], preferred_element_type=jnp.float32)


def _mm2(h, wl, wr):
    _, n, din = h.shape
    dout = wl.shape[2]
    return pl.pallas_call(
        _mm2_body,
        grid=(2,),
        in_specs=[
            pl.BlockSpec((1, n, din), lambda s: (s, 0, 0)),
            pl.BlockSpec((1, din, dout), lambda s: (s, 0, 0)),
            pl.BlockSpec((1, din, dout), lambda s: (s, 0, 0)),
        ],
        out_specs=[
            pl.BlockSpec((1, n, dout), lambda s: (s, 0, 0)),
            pl.BlockSpec((1, n, dout), lambda s: (s, 0, 0)),
        ],
        out_shape=[
            jax.ShapeDtypeStruct((2, n, dout), jnp.float32),
            jax.ShapeDtypeStruct((2, n, dout), jnp.float32),
        ],
    )(h, wl, wr)


def _node_body(acc_ref, den_ref, b_ref, g_ref, bb_ref, batch_ref,
               h_ref, pool_ref):
    acc = acc_ref[0]
    den = den_ref[0][:, 0:1]
    out = acc / (den + 1e-16) + b_ref[0]
    r = jnp.maximum(out, 0.0)
    mu = jnp.mean(r, axis=0, keepdims=True)
    var = jnp.mean((r - mu) ** 2, axis=0, keepdims=True)
    h = (r - mu) / jnp.sqrt(var + 1e-5) * g_ref[0] + bb_ref[0]
    h_ref[0] = h
    bvec = batch_ref[0, 0]
    seg = lax.broadcasted_iota(jnp.int32, (N_GRAPHS, bvec.shape[0]), 0)
    p = (seg == bvec[None, :]).astype(jnp.float32)
    pool_ref[0] = jnp.dot(p, h, preferred_element_type=jnp.float32)


def _node(acc, den, b, g, bb, batch):
    _, n, dout = acc.shape
    return pl.pallas_call(
        _node_body,
        grid=(2,),
        in_specs=[
            pl.BlockSpec((1, n, dout), lambda s: (s, 0, 0)),
            pl.BlockSpec((1, n, den.shape[2]), lambda s: (s, 0, 0)),
            pl.BlockSpec((1, 1, dout), lambda s: (s, 0, 0)),
            pl.BlockSpec((1, 1, dout), lambda s: (s, 0, 0)),
            pl.BlockSpec((1, 1, dout), lambda s: (s, 0, 0)),
            pl.BlockSpec((1, 1, n), lambda s: (s, 0, 0)),
        ],
        out_specs=[
            pl.BlockSpec((1, n, dout), lambda s: (s, 0, 0)),
            pl.BlockSpec((1, N_GRAPHS, dout), lambda s: (s, 0, 0)),
        ],
        out_shape=[
            jax.ShapeDtypeStruct((2, n, dout), jnp.float32),
            jax.ShapeDtypeStruct((2, N_GRAPHS, dout), jnp.float32),
        ],
    )(acc, den, b, g, bb, batch)


def _bn_rows(x, g, b):
    mu = jnp.mean(x, axis=0, keepdims=True)
    var = jnp.mean((x - mu) ** 2, axis=0, keepdims=True)
    return (x - mu) / jnp.sqrt(var + 1e-5) * g + b


def _head_body(hin_ref, w1_ref, b1_ref, g1_ref, bb1_ref,
               w2_ref, b2_ref, g2_ref, bb2_ref, w3_ref, b3_ref,
               sig_ref, lsm_ref):
    h = hin_ref[...]
    h = jnp.maximum(jnp.dot(h, w1_ref[...],
                            preferred_element_type=jnp.float32) + b1_ref[...], 0.0)
    h = _bn_rows(h, g1_ref[...], bb1_ref[...])
    h = jnp.maximum(jnp.dot(h, w2_ref[...],
                            preferred_element_type=jnp.float32) + b2_ref[...], 0.0)
    h = _bn_rows(h, g2_ref[...], bb2_ref[...])
    h = jnp.dot(h, w3_ref[...], preferred_element_type=jnp.float32) + b3_ref[...]
    sig_ref[...] = 1.0 / (1.0 + jnp.exp(-h))
    m = jnp.max(h, axis=1, keepdims=True)
    ex = jnp.exp(h - m)
    lsm_ref[...] = (h - m) - jnp.log(jnp.sum(ex, axis=1, keepdims=True))


def _head(hin, p):
    d_out = p["lin3"]["W"].shape[1]
    args = [hin,
            p["lin1"]["W"], p["lin1"]["b"][None, :], p["m1"]["g"][None, :],
            p["m1"]["b"][None, :],
            p["lin2"]["W"], p["lin2"]["b"][None, :], p["m2"]["g"][None, :],
            p["m2"]["b"][None, :],
            p["lin3"]["W"], p["lin3"]["b"][None, :]]
    return pl.pallas_call(
        _head_body,
        out_shape=[jax.ShapeDtypeStruct((N_GRAPHS, d_out), jnp.float32),
                   jax.ShapeDtypeStruct((N_GRAPHS, d_out), jnp.float32)],
    )(*args)


def _edge_stage(xl, xr, src, dst, att):
    def one(xl_s, xr_s, src_s, dst_s, att_s):
        e = jnp.maximum(xl_s[src_s] + xr_s[dst_s],
                        0.2 * (xl_s[src_s] + xr_s[dst_s])) @ att_s
        ex = jnp.exp(e)
        den = jax.ops.segment_sum(ex, dst_s, num_segments=N_NODES)
        acc = jax.ops.segment_sum(ex[:, None] * xl_s[src_s], dst_s,
                                  num_segments=N_NODES)
        return acc, jnp.broadcast_to(den[:, None], (N_NODES, 16))
    acc0, den0 = one(xl[0], xr[0], src[0], dst[0], att[0])
    acc1, den1 = one(xl[1], xr[1], src[1], dst[1], att[1])
    return jnp.stack([acc0, acc1]), jnp.stack([den0, den1])


def kernel(x0, x1, edge_index0, edge_index1, batch0, batch1, params, train):
    src = jnp.stack([edge_index0[0], edge_index1[0]])
    dst = jnp.stack([edge_index0[1], edge_index1[1]])
    batch = jnp.stack([batch0, batch1])[:, None, :]
    h = jnp.stack([x0, x1])
    streams = params["streams"]
    pooled = []
    for l in range(4):
        wl = jnp.stack([streams[s]["gat"][l]["Wl"] for s in range(2)])
        wr = jnp.stack([streams[s]["gat"][l]["Wr"] for s in range(2)])
        att = jnp.stack([streams[s]["gat"][l]["att"] for s in range(2)])
        b = jnp.stack([streams[s]["gat"][l]["b"] for s in range(2)])[:, None, :]
        g = jnp.stack([streams[s]["bn"][l]["g"] for s in range(2)])[:, None, :]
        bb = jnp.stack([streams[s]["bn"][l]["b"] for s in range(2)])[:, None, :]
        xl, xr = _mm2(h, wl, wr)
        acc, den = _edge_stage(xl, xr, src, dst, att)
        h, pool = _node(acc, den, b, g, bb, batch)
        pooled.append(pool)
    s0 = jnp.concatenate([p[0] for p in pooled], axis=1)
    s1 = jnp.concatenate([p[1] for p in pooled], axis=1)
    hin = jnp.concatenate([s0, s1], axis=1)
    sig, lsm = _head(hin, params)
    return sig, lsm


# speedup vs baseline: 9.4967x; 3.9246x over previous
import functools

import jax
import jax.numpy as jnp
from jax import lax
from jax.experimental import pallas as pl
from jax.experimental.pallas import tpu as pltpu
from jax.experimental.pallas import tpu_sc as plsc

N_NODES = 10000
N_GRAPHS = 64

_NT = 16
N_PAD = 10240
_ROWS_PT = N_PAD // _NT
_EPT = 20096
E_PAD = _EPT * _NT


def _mm2_body(h_ref, wl_ref, wr_ref, xl_ref, xr_ref):
    h = h_ref[0]
    xl_ref[0] = jnp.dot(h, wl_ref[0], preferred_element_type=jnp.float32)
    xr_ref[0] = jnp.dot(h, wr_ref[0], preferred_element_type=jnp.float32)


def _mm2(h, wl, wr):
    _, n, din = h.shape
    dout = wl.shape[2]
    return pl.pallas_call(
        _mm2_body,
        grid=(2,),
        in_specs=[
            pl.BlockSpec((1, n, din), lambda s: (s, 0, 0)),
            pl.BlockSpec((1, din, dout), lambda s: (s, 0, 0)),
            pl.BlockSpec((1, din, dout), lambda s: (s, 0, 0)),
        ],
        out_specs=[
            pl.BlockSpec((1, n, dout), lambda s: (s, 0, 0)),
            pl.BlockSpec((1, n, dout), lambda s: (s, 0, 0)),
        ],
        out_shape=[
            jax.ShapeDtypeStruct((2, n, dout), jnp.float32),
            jax.ShapeDtypeStruct((2, n, dout), jnp.float32),
        ],
    )(h, wl, wr)


def _node_body(accden_ref, b_ref, g_ref, bb_ref, batch_ref,
               h_ref, pool_ref):
    dout = b_ref.shape[2]
    acc = accden_ref[0, :N_NODES, :dout]
    den = accden_ref[0, :N_NODES, dout:dout + 1]
    out = acc / (den + 1e-16) + b_ref[0]
    r = jnp.maximum(out, 0.0)
    mu = jnp.mean(r, axis=0, keepdims=True)
    var = jnp.mean((r - mu) ** 2, axis=0, keepdims=True)
    h = (r - mu) / jnp.sqrt(var + 1e-5) * g_ref[0] + bb_ref[0]
    h_ref[0] = h
    bvec = batch_ref[0, 0]
    seg = lax.broadcasted_iota(jnp.int32, (N_GRAPHS, bvec.shape[0]), 0)
    p = (seg == bvec[None, :]).astype(jnp.float32)
    pool_ref[0] = jnp.dot(p, h, preferred_element_type=jnp.float32)


def _node(accden, b, g, bb, batch):
    _, npad, dw = accden.shape
    dout = dw - 16
    n = N_NODES
    return pl.pallas_call(
        _node_body,
        grid=(2,),
        in_specs=[
            pl.BlockSpec((1, npad, dw), lambda s: (s, 0, 0)),
            pl.BlockSpec((1, 1, dout), lambda s: (s, 0, 0)),
            pl.BlockSpec((1, 1, dout), lambda s: (s, 0, 0)),
            pl.BlockSpec((1, 1, dout), lambda s: (s, 0, 0)),
            pl.BlockSpec((1, 1, n), lambda s: (s, 0, 0)),
        ],
        out_specs=[
            pl.BlockSpec((1, n, dout), lambda s: (s, 0, 0)),
            pl.BlockSpec((1, N_GRAPHS, dout), lambda s: (s, 0, 0)),
        ],
        out_shape=[
            jax.ShapeDtypeStruct((2, n, dout), jnp.float32),
            jax.ShapeDtypeStruct((2, N_GRAPHS, dout), jnp.float32),
        ],
    )(accden, b, g, bb, batch)


def _bn_rows(x, g, b):
    mu = jnp.mean(x, axis=0, keepdims=True)
    var = jnp.mean((x - mu) ** 2, axis=0, keepdims=True)
    return (x - mu) / jnp.sqrt(var + 1e-5) * g + b


def _head_body(hin_ref, w1_ref, b1_ref, g1_ref, bb1_ref,
               w2_ref, b2_ref, g2_ref, bb2_ref, w3_ref, b3_ref,
               sig_ref, lsm_ref):
    h = hin_ref[...]
    h = jnp.maximum(jnp.dot(h, w1_ref[...],
                            preferred_element_type=jnp.float32) + b1_ref[...], 0.0)
    h = _bn_rows(h, g1_ref[...], bb1_ref[...])
    h = jnp.maximum(jnp.dot(h, w2_ref[...],
                            preferred_element_type=jnp.float32) + b2_ref[...], 0.0)
    h = _bn_rows(h, g2_ref[...], bb2_ref[...])
    h = jnp.dot(h, w3_ref[...], preferred_element_type=jnp.float32) + b3_ref[...]
    sig_ref[...] = 1.0 / (1.0 + jnp.exp(-h))
    m = jnp.max(h, axis=1, keepdims=True)
    ex = jnp.exp(h - m)
    lsm_ref[...] = (h - m) - jnp.log(jnp.sum(ex, axis=1, keepdims=True))


def _head(hin, p):
    d_out = p["lin3"]["W"].shape[1]
    args = [hin,
            p["lin1"]["W"], p["lin1"]["b"][None, :], p["m1"]["g"][None, :],
            p["m1"]["b"][None, :],
            p["lin2"]["W"], p["lin2"]["b"][None, :], p["m2"]["g"][None, :],
            p["m2"]["b"][None, :],
            p["lin3"]["W"], p["lin3"]["b"][None, :]]
    return pl.pallas_call(
        _head_body,
        out_shape=[jax.ShapeDtypeStruct((N_GRAPHS, d_out), jnp.float32),
                   jax.ShapeDtypeStruct((N_GRAPHS, d_out), jnp.float32)],
    )(*args)


def _allsum16(v):
    lanes = lax.iota(jnp.int32, 16)
    for k in (8, 4, 2, 1):
        v = v + v.at[lanes ^ k].get(mode="promise_in_bounds")
    return v


@functools.lru_cache(maxsize=None)
def _edge_sc(dout):
    nj = dout // 16
    dw = dout + 16
    chunk = 64 if dout >= 128 else 128
    nchunk = _EPT // chunk
    mesh = plsc.VectorSubcoreMesh(core_axis_name="c", subcore_axis_name="s")

    @functools.partial(
        pl.kernel, mesh=mesh,
        compiler_params=pltpu.CompilerParams(use_tc_tiling_on_sc=False),
        out_type=jax.ShapeDtypeStruct((2, N_PAD, dw), jnp.float32),
        scratch_types=[
            pltpu.VMEM((chunk,), jnp.int32),
            pltpu.VMEM((chunk,), jnp.int32),
            pltpu.VMEM((chunk,), jnp.int32),
            pltpu.VMEM((chunk, dout), jnp.float32),
            pltpu.VMEM((chunk, dout), jnp.float32),
            pltpu.VMEM((chunk, dw), jnp.float32),
            pltpu.VMEM((dout,), jnp.float32),
            pltpu.VMEM_SHARED((N_PAD, dw), jnp.float32),
            pltpu.SemaphoreType.DMA,
            pltpu.SemaphoreType.DMA,
        ])
    def k(xl_hbm, xr_hbm, srcg_hbm, dstg_hbm, dstl_hbm, att_hbm,
          acc_out,
          sidx, didx, lidx, bufL, bufR, wbuf, attbuf,
          acc_sh, sem1, sem2):
        c = lax.axis_index("c")
        s = lax.axis_index("s")
        pltpu.sync_copy(att_hbm.at[c], attbuf)

        zero16 = jnp.zeros((16,), jnp.float32)

        def _zrow(i, _):
            for j in range(nj + 1):
                wbuf[i, pl.ds(j * 16, 16)] = zero16
            return 0

        lax.fori_loop(0, chunk, _zrow, 0)
        for kk in range(_ROWS_PT // chunk):
            r0 = s * _ROWS_PT + kk * chunk
            pltpu.sync_copy(wbuf, acc_sh.at[pl.ds(r0, chunk)])
        plsc.subcore_barrier()

        def _chunk(g, _):
            base = s * _EPT + g * chunk
            pltpu.sync_copy(srcg_hbm.at[c, pl.ds(base, chunk)], sidx)
            pltpu.sync_copy(dstg_hbm.at[c, pl.ds(base, chunk)], didx)
            pltpu.sync_copy(dstl_hbm.at[c, pl.ds(base, chunk)], lidx)
            cp1 = pltpu.async_copy(xl_hbm.at[sidx], bufL, sem1)
            cp2 = pltpu.async_copy(xr_hbm.at[didx], bufR, sem2)
            cp1.wait()
            cp2.wait()

            def _edge(e, _):
                avs = []
                acc16 = zero16
                for j in range(nj):
                    a = bufL[e, pl.ds(j * 16, 16)]
                    b = bufR[e, pl.ds(j * 16, 16)]
                    v = a + b
                    t = jnp.maximum(v, v * 0.2)
                    acc16 = acc16 + t * attbuf[pl.ds(j * 16, 16)]
                    avs.append(a)
                ex = jnp.exp(_allsum16(acc16))
                for j in range(nj):
                    wbuf[e, pl.ds(j * 16, 16)] = avs[j] * ex
                wbuf[e, pl.ds(nj * 16, 16)] = ex
                return 0

            lax.fori_loop(0, chunk, _edge, 0)
            pltpu.sync_copy(wbuf, acc_sh.at[lidx], add=True)
            return 0

        lax.fori_loop(0, nchunk, _chunk, 0)
        plsc.subcore_barrier()
        for kk in range(_ROWS_PT // chunk):
            r0 = s * _ROWS_PT + kk * chunk
            pltpu.sync_copy(acc_sh.at[pl.ds(r0, chunk)],
                            acc_out.at[c, pl.ds(r0, chunk)])

    return k


def _edge_stage(xl, xr, src_g, dst_g, dst_l, att):
    dout = xl.shape[2]
    xl2 = xl.reshape(2 * N_NODES, dout)
    xr2 = xr.reshape(2 * N_NODES, dout)
    return _edge_sc(dout)(xl2, xr2, src_g, dst_g, dst_l, att)


def kernel(x0, x1, edge_index0, edge_index1, batch0, batch1, params, train):
    e = edge_index0.shape[1]
    pad = E_PAD - e
    zpad = jnp.zeros((pad,), jnp.int32)
    gpad = jnp.full((pad,), N_PAD - 1, jnp.int32)
    src_g = jnp.stack([jnp.concatenate([edge_index0[0], zpad]),
                       jnp.concatenate([edge_index1[0] + N_NODES, zpad])])
    dst_g = jnp.stack([jnp.concatenate([edge_index0[1], zpad]),
                       jnp.concatenate([edge_index1[1] + N_NODES, zpad])])
    dst_l = jnp.stack([jnp.concatenate([edge_index0[1], gpad]),
                       jnp.concatenate([edge_index1[1], gpad])])
    batch = jnp.stack([batch0, batch1])[:, None, :]
    h = jnp.stack([x0, x1])
    streams = params["streams"]
    pooled = []
    for l in range(4):
        wl = jnp.stack([streams[s]["gat"][l]["Wl"] for s in range(2)])
        wr = jnp.stack([streams[s]["gat"][l]["Wr"] for s in range(2)])
        att = jnp.stack([streams[s]["gat"][l]["att"] for s in range(2)])
        b = jnp.stack([streams[s]["gat"][l]["b"] for s in range(2)])[:, None, :]
        g = jnp.stack([streams[s]["bn"][l]["g"] for s in range(2)])[:, None, :]
        bb = jnp.stack([streams[s]["bn"][l]["b"] for s in range(2)])[:, None, :]
        xl, xr = _mm2(h, wl, wr)
        accden = _edge_stage(xl, xr, src_g, dst_g, dst_l, att)
        h, pool = _node(accden, b, g, bb, batch)
        pooled.append(pool)
    s0 = jnp.concatenate([p[0] for p in pooled], axis=1)
    s1 = jnp.concatenate([p[1] for p in pooled], axis=1)
    hin = jnp.concatenate([s0, s1], axis=1)
    sig, lsm = _head(hin, params)
    return sig, lsm


# speedup vs baseline: 13.3669x; 1.4075x over previous
import functools

import jax
import jax.numpy as jnp
from jax import lax
from jax.experimental import pallas as pl
from jax.experimental.pallas import tpu as pltpu
from jax.experimental.pallas import tpu_sc as plsc

N_NODES = 10000
N_GRAPHS = 64

_NT = 16
N_PAD = 10240
_ROWS_PT = N_PAD // _NT
_EPT = 20480
E_PAD = _EPT * _NT
_SBC = 8


def _mm2_body(h_ref, wl_ref, wr_ref, xl_ref, xr_ref):
    h = h_ref[0]
    xl_ref[0] = jnp.dot(h, wl_ref[0], preferred_element_type=jnp.float32)
    xr_ref[0] = jnp.dot(h, wr_ref[0], preferred_element_type=jnp.float32)


def _mm2(h, wl, wr):
    _, n, din = h.shape
    dout = wl.shape[2]
    return pl.pallas_call(
        _mm2_body,
        grid=(2,),
        in_specs=[
            pl.BlockSpec((1, n, din), lambda s: (s, 0, 0)),
            pl.BlockSpec((1, din, dout), lambda s: (s, 0, 0)),
            pl.BlockSpec((1, din, dout), lambda s: (s, 0, 0)),
        ],
        out_specs=[
            pl.BlockSpec((1, n, dout), lambda s: (s, 0, 0)),
            pl.BlockSpec((1, n, dout), lambda s: (s, 0, 0)),
        ],
        out_shape=[
            jax.ShapeDtypeStruct((2, n, dout), jnp.float32),
            jax.ShapeDtypeStruct((2, n, dout), jnp.float32),
        ],
    )(h, wl, wr)


def _node_body(accden_ref, b_ref, g_ref, bb_ref, batch_ref,
               h_ref, pool_ref):
    dout = b_ref.shape[2]
    acc = accden_ref[0, :N_NODES, :dout]
    den = accden_ref[0, :N_NODES, dout:dout + 1]
    out = acc / (den + 1e-16) + b_ref[0]
    r = jnp.maximum(out, 0.0)
    mu = jnp.mean(r, axis=0, keepdims=True)
    var = jnp.mean((r - mu) ** 2, axis=0, keepdims=True)
    h = (r - mu) / jnp.sqrt(var + 1e-5) * g_ref[0] + bb_ref[0]
    h_ref[0] = h
    bvec = batch_ref[0, 0]
    seg = lax.broadcasted_iota(jnp.int32, (N_GRAPHS, bvec.shape[0]), 0)
    p = (seg == bvec[None, :]).astype(jnp.float32)
    pool_ref[0] = jnp.dot(p, h, preferred_element_type=jnp.float32)


def _node(accden, b, g, bb, batch):
    _, npad, dw = accden.shape
    dout = dw - 16
    n = N_NODES
    return pl.pallas_call(
        _node_body,
        grid=(2,),
        in_specs=[
            pl.BlockSpec((1, npad, dw), lambda s: (s, 0, 0)),
            pl.BlockSpec((1, 1, dout), lambda s: (s, 0, 0)),
            pl.BlockSpec((1, 1, dout), lambda s: (s, 0, 0)),
            pl.BlockSpec((1, 1, dout), lambda s: (s, 0, 0)),
            pl.BlockSpec((1, 1, n), lambda s: (s, 0, 0)),
        ],
        out_specs=[
            pl.BlockSpec((1, n, dout), lambda s: (s, 0, 0)),
            pl.BlockSpec((1, N_GRAPHS, dout), lambda s: (s, 0, 0)),
        ],
        out_shape=[
            jax.ShapeDtypeStruct((2, n, dout), jnp.float32),
            jax.ShapeDtypeStruct((2, N_GRAPHS, dout), jnp.float32),
        ],
    )(accden, b, g, bb, batch)


def _bn_rows(x, g, b):
    mu = jnp.mean(x, axis=0, keepdims=True)
    var = jnp.mean((x - mu) ** 2, axis=0, keepdims=True)
    return (x - mu) / jnp.sqrt(var + 1e-5) * g + b


def _head_body(hin_ref, w1_ref, b1_ref, g1_ref, bb1_ref,
               w2_ref, b2_ref, g2_ref, bb2_ref, w3_ref, b3_ref,
               sig_ref, lsm_ref):
    h = hin_ref[...]
    h = jnp.maximum(jnp.dot(h, w1_ref[...],
                            preferred_element_type=jnp.float32) + b1_ref[...], 0.0)
    h = _bn_rows(h, g1_ref[...], bb1_ref[...])
    h = jnp.maximum(jnp.dot(h, w2_ref[...],
                            preferred_element_type=jnp.float32) + b2_ref[...], 0.0)
    h = _bn_rows(h, g2_ref[...], bb2_ref[...])
    h = jnp.dot(h, w3_ref[...], preferred_element_type=jnp.float32) + b3_ref[...]
    sig_ref[...] = 1.0 / (1.0 + jnp.exp(-h))
    m = jnp.max(h, axis=1, keepdims=True)
    ex = jnp.exp(h - m)
    lsm_ref[...] = (h - m) - jnp.log(jnp.sum(ex, axis=1, keepdims=True))


def _head(hin, p):
    d_out = p["lin3"]["W"].shape[1]
    args = [hin,
            p["lin1"]["W"], p["lin1"]["b"][None, :], p["m1"]["g"][None, :],
            p["m1"]["b"][None, :],
            p["lin2"]["W"], p["lin2"]["b"][None, :], p["m2"]["g"][None, :],
            p["m2"]["b"][None, :],
            p["lin3"]["W"], p["lin3"]["b"][None, :]]
    return pl.pallas_call(
        _head_body,
        out_shape=[jax.ShapeDtypeStruct((N_GRAPHS, d_out), jnp.float32),
                   jax.ShapeDtypeStruct((N_GRAPHS, d_out), jnp.float32)],
    )(*args)


def _allsum16(v):
    lanes = lax.iota(jnp.int32, 16)
    for k in (8, 4, 2, 1):
        v = v + v.at[lanes ^ k].get(mode="promise_in_bounds")
    return v


@functools.lru_cache(maxsize=None)
def _edge_sc(dout):
    nj = dout // 16
    dw = dout + 16
    chunk = 40 if dout >= 128 else 128
    nchunk = _EPT // chunk
    nzc = _ROWS_PT // chunk if _ROWS_PT % chunk == 0 else None
    mesh = plsc.VectorSubcoreMesh(core_axis_name="c", subcore_axis_name="s")

    @functools.partial(
        pl.kernel, mesh=mesh,
        compiler_params=pltpu.CompilerParams(use_tc_tiling_on_sc=False),
        out_type=jax.ShapeDtypeStruct((2, N_PAD, dw), jnp.float32),
        scratch_types=[
            pltpu.VMEM((2, _SBC, chunk), jnp.int32),
            pltpu.VMEM((2, _SBC, chunk), jnp.int32),
            pltpu.VMEM((2, _SBC, chunk), jnp.int32),
            pltpu.VMEM((2, chunk, dout), jnp.float32),
            pltpu.VMEM((2, chunk, dout), jnp.float32),
            pltpu.VMEM((2, chunk, dw), jnp.float32),
            pltpu.VMEM((dout,), jnp.float32),
            pltpu.VMEM_SHARED((N_PAD, dw), jnp.float32),
            pltpu.SemaphoreType.DMA,
            pltpu.SemaphoreType.DMA,
            pltpu.SemaphoreType.DMA,
        ])
    def k(xl_hbm, xr_hbm, srcg_hbm, dstg_hbm, dstl_hbm, att_hbm,
          acc_out,
          sidxs, didxs, lidxs, bufL, bufR, wbuf, attbuf,
          acc_sh, semL, semR, semS):
        c = lax.axis_index("c")
        s = lax.axis_index("s")
        pltpu.sync_copy(att_hbm.at[c], attbuf)
        att_vs = [attbuf[pl.ds(j * 16, 16)] for j in range(nj)]

        zero16 = jnp.zeros((16,), jnp.float32)

        def _zrow(i, _):
            for j in range(nj + 1):
                wbuf[0, i, pl.ds(j * 16, 16)] = zero16
            return 0

        lax.fori_loop(0, chunk, _zrow, 0)
        for kk in range(nzc):
            r0 = s * _ROWS_PT + kk * chunk
            pltpu.sync_copy(wbuf.at[0], acc_sh.at[pl.ds(r0, chunk)])
        plsc.subcore_barrier()

        row0 = s * nchunk

        def _load_super(su):
            sl = su & 1
            base = row0 + su * _SBC
            pltpu.sync_copy(srcg_hbm.at[c, pl.ds(base, _SBC)], sidxs.at[sl])
            pltpu.sync_copy(dstg_hbm.at[c, pl.ds(base, _SBC)], didxs.at[sl])
            pltpu.sync_copy(dstl_hbm.at[c, pl.ds(base, _SBC)], lidxs.at[sl])

        def _gather_refs(a, b):
            su = a // _SBC
            j = a - su * _SBC
            sl = su & 1
            cpL = pltpu.make_async_copy(xl_hbm.at[sidxs.at[sl, j]],
                                        bufL.at[b], semL)
            cpR = pltpu.make_async_copy(xr_hbm.at[didxs.at[sl, j]],
                                        bufR.at[b], semR)
            return cpL, cpR

        def _scatter_ref(a, b):
            su = a // _SBC
            j = a - su * _SBC
            sl = su & 1
            return pltpu.make_async_copy(wbuf.at[b], acc_sh.at[lidxs.at[sl, j]],
                                         semS)

        _load_super(0)
        cpL0, cpR0 = _gather_refs(0, 0)
        cpL0.start()
        cpR0.start()

        def _iter(a, _):
            b = a & 1
            an = a + 1
            su_n = an // _SBC

            @pl.when((an < nchunk) & (an == su_n * _SBC))
            def _():
                _load_super(su_n)

            @pl.when(an < nchunk)
            def _():
                cpL, cpR = _gather_refs(an, 1 - b)
                cpL.start()
                cpR.start()

            cpL, cpR = _gather_refs(a, b)
            cpL.wait()
            cpR.wait()

            @pl.when(a >= 2)
            def _():
                _scatter_ref(a - 2, b).wait()

            def _edge(e, _):
                avs = []
                acc16 = zero16
                for j in range(nj):
                    va = bufL[b, e, pl.ds(j * 16, 16)]
                    vb = bufR[b, e, pl.ds(j * 16, 16)]
                    v = va + vb
                    t = jnp.maximum(v, v * 0.2)
                    acc16 = acc16 + t * att_vs[j]
                    avs.append(va)
                ex = jnp.exp(_allsum16(acc16))
                for j in range(nj):
                    wbuf[b, e, pl.ds(j * 16, 16)] = avs[j] * ex
                wbuf[b, e, pl.ds(nj * 16, 16)] = ex
                return 0

            lax.fori_loop(0, chunk, _edge, 0)
            _scatter_ref(a, b).start(add=True)
            return 0

        lax.fori_loop(0, nchunk, _iter, 0)
        _scatter_ref(nchunk - 2, (nchunk - 2) & 1).wait()
        _scatter_ref(nchunk - 1, (nchunk - 1) & 1).wait()
        plsc.subcore_barrier()
        for kk in range(nzc):
            r0 = s * _ROWS_PT + kk * chunk
            pltpu.sync_copy(acc_sh.at[pl.ds(r0, chunk)],
                            acc_out.at[c, pl.ds(r0, chunk)])

    return k


def _edge_stage(xl, xr, src_g, dst_g, dst_l, att):
    dout = xl.shape[2]
    chunk = 40 if dout >= 128 else 128
    xl2 = xl.reshape(2 * N_NODES, dout)
    xr2 = xr.reshape(2 * N_NODES, dout)
    s3 = src_g.reshape(2, E_PAD // chunk, chunk)
    d3 = dst_g.reshape(2, E_PAD // chunk, chunk)
    l3 = dst_l.reshape(2, E_PAD // chunk, chunk)
    return _edge_sc(dout)(xl2, xr2, s3, d3, l3, att)


def kernel(x0, x1, edge_index0, edge_index1, batch0, batch1, params, train):
    e = edge_index0.shape[1]
    pad = E_PAD - e
    zpad = jnp.zeros((pad,), jnp.int32)
    gpad = jnp.full((pad,), N_PAD - 1, jnp.int32)
    src_g = jnp.stack([jnp.concatenate([edge_index0[0], zpad]),
                       jnp.concatenate([edge_index1[0] + N_NODES, zpad])])
    dst_g = jnp.stack([jnp.concatenate([edge_index0[1], zpad]),
                       jnp.concatenate([edge_index1[1] + N_NODES, zpad])])
    dst_l = jnp.stack([jnp.concatenate([edge_index0[1], gpad]),
                       jnp.concatenate([edge_index1[1], gpad])])
    batch = jnp.stack([batch0, batch1])[:, None, :]
    h = jnp.stack([x0, x1])
    streams = params["streams"]
    pooled = []
    for l in range(4):
        wl = jnp.stack([streams[s]["gat"][l]["Wl"] for s in range(2)])
        wr = jnp.stack([streams[s]["gat"][l]["Wr"] for s in range(2)])
        att = jnp.stack([streams[s]["gat"][l]["att"] for s in range(2)])
        b = jnp.stack([streams[s]["gat"][l]["b"] for s in range(2)])[:, None, :]
        g = jnp.stack([streams[s]["bn"][l]["g"] for s in range(2)])[:, None, :]
        bb = jnp.stack([streams[s]["bn"][l]["b"] for s in range(2)])[:, None, :]
        xl, xr = _mm2(h, wl, wr)
        accden = _edge_stage(xl, xr, src_g, dst_g, dst_l, att)
        h, pool = _node(accden, b, g, bb, batch)
        pooled.append(pool)
    s0 = jnp.concatenate([p[0] for p in pooled], axis=1)
    s1 = jnp.concatenate([p[1] for p in pooled], axis=1)
    hin = jnp.concatenate([s0, s1], axis=1)
    sig, lsm = _head(hin, params)
    return sig, lsm
